# baseline (speedup 1.0000x reference)
import functools

import jax
import jax.numpy as jnp
from jax import lax
from jax.experimental import pallas as pl
from jax.experimental.pallas import tpu as pltpu
from jax.experimental.pallas import tpu_sc as plsc

_N_CORES = 2
_N_SUB = 16
_N_GRAPHS = 64
_DEG_CHUNK = 2000
_DEG_W = 16


def _make_sc_degree(n_pad, n_edges):
    ntiles = _N_CORES * _N_SUB
    e_per_tile = n_edges // ntiles
    n_chunks = e_per_tile // _DEG_CHUNK
    rps = n_pad // _N_SUB

    mesh = plsc.VectorSubcoreMesh(core_axis_name="c", subcore_axis_name="s")

    @functools.partial(
        pl.kernel,
        out_type=jax.ShapeDtypeStruct((_N_CORES, n_pad, _DEG_W), jnp.float32),
        mesh=mesh,
        compiler_params=pltpu.CompilerParams(use_tc_tiling_on_sc=False),
        scratch_types=[
            pltpu.VMEM((n_chunks, _DEG_CHUNK), jnp.int32),
            pltpu.VMEM((_DEG_CHUNK, _DEG_W), jnp.float32),
            pltpu.VMEM_SHARED((n_pad, _DEG_W), jnp.float32),
        ],
    )
    def deg_kernel(dst_hbm, ones_hbm, zero_hbm, out_hbm, didx, ones_v, acc):
        c = lax.axis_index("c")
        s = lax.axis_index("s")
        tid = c * _N_SUB + s
        pltpu.sync_copy(dst_hbm.at[tid], didx)
        pltpu.sync_copy(ones_hbm, ones_v)
        pltpu.sync_copy(zero_hbm.at[pl.ds(s * rps, rps)],
                        acc.at[pl.ds(s * rps, rps)])
        plsc.subcore_barrier()

        def body(i, carry):
            pltpu.sync_copy(ones_v, acc.at[didx.at[i]], add=True)
            return carry

        lax.fori_loop(0, n_chunks, body, 0)
        plsc.subcore_barrier()
        pltpu.sync_copy(acc.at[pl.ds(s * rps, rps)],
                        out_hbm.at[c, pl.ds(s * rps, rps)])

    return deg_kernel


def _make_sc_scatter(n_pad, n_edges, d, chunk):
    ntiles = _N_CORES * _N_SUB
    e_per_tile = n_edges // ntiles
    n_chunks = e_per_tile // chunk
    rps = n_pad // _N_SUB

    mesh = plsc.VectorSubcoreMesh(core_axis_name="c", subcore_axis_name="s")

    @functools.partial(
        pl.kernel,
        out_type=jax.ShapeDtypeStruct((_N_CORES, n_pad, d), jnp.float32),
        mesh=mesh,
        compiler_params=pltpu.CompilerParams(use_tc_tiling_on_sc=False),
        scratch_types=[
            pltpu.VMEM((n_chunks, chunk), jnp.int32),
            pltpu.VMEM((n_chunks, chunk), jnp.int32),
            pltpu.VMEM((2, chunk, d), jnp.float32),
            pltpu.VMEM_SHARED((n_pad, d), jnp.float32),
            pltpu.SemaphoreType.DMA((2,)),
            pltpu.SemaphoreType.DMA((2,)),
        ],
    )
    def scatter_kernel(src_hbm, dst_hbm, feat_hbm, zero_hbm, out_hbm,
                       sidx, didx, rows, acc, gsem, ssem):
        c = lax.axis_index("c")
        s = lax.axis_index("s")
        tid = c * _N_SUB + s
        pltpu.sync_copy(src_hbm.at[tid], sidx)
        pltpu.sync_copy(dst_hbm.at[tid], didx)
        pltpu.sync_copy(zero_hbm.at[pl.ds(s * rps, rps)],
                        acc.at[pl.ds(s * rps, rps)])
        plsc.subcore_barrier()
        pltpu.async_copy(feat_hbm.at[sidx.at[0]], rows.at[0], gsem.at[0])

        def body(i, carry):
            b = lax.rem(i, 2)
            nb = lax.rem(i + 1, 2)

            @pl.when(i >= 1)
            def _drain_prev_scatter():
                pltpu.make_async_copy(rows.at[nb], acc.at[didx.at[i - 1]],
                                      ssem.at[nb]).wait()

            @pl.when(i + 1 < n_chunks)
            def _fire_next_gather():
                pltpu.async_copy(feat_hbm.at[sidx.at[i + 1]], rows.at[nb],
                                 gsem.at[nb])

            pltpu.make_async_copy(feat_hbm.at[sidx.at[i]], rows.at[b],
                                  gsem.at[b]).wait()
            pltpu.async_copy(rows.at[b], acc.at[didx.at[i]], ssem.at[b],
                             add=True)
            return carry

        lax.fori_loop(0, n_chunks, body, 0)
        last = (n_chunks - 1) % 2
        pltpu.make_async_copy(rows.at[last], acc.at[didx.at[n_chunks - 1]],
                              ssem.at[last]).wait()
        plsc.subcore_barrier()
        pltpu.sync_copy(acc.at[pl.ds(s * rps, rps)],
                        out_hbm.at[c, pl.ds(s * rps, rps)])

    return scatter_kernel


def _tc1_body(degp_ref, x_ref, w1_ref, dinv_ref, hs_ref):
    deg = degp_ref[0][:, 0:1] + degp_ref[1][:, 0:1] + 1.0
    dinv = lax.rsqrt(deg)
    h = jnp.dot(x_ref[...], w1_ref[...], preferred_element_type=jnp.float32)
    dinv_ref[...] = dinv
    hs_ref[...] = h * dinv


def _tc2_body(p_ref, hs1_ref, dinv_ref, b1_ref, w2_ref, hs2_ref):
    ssum = p_ref[0] + p_ref[1] + hs1_ref[...]
    dinv = dinv_ref[...]
    h1 = jnp.maximum(ssum * dinv + b1_ref[...], 0.0)
    hs2_ref[...] = jnp.dot(h1, w2_ref[...],
                           preferred_element_type=jnp.float32) * dinv


def _tc3_body(p_ref, hs2_ref, dinv_ref, b2_ref, batch_ref, fc1w_ref,
              fc1b_ref, fc2w_ref, fc2b_ref, out_ref, sums, counts, *,
              n_blocks, blk):
    i = pl.program_id(0)

    @pl.when(i == 0)
    def _init():
        sums[...] = jnp.zeros_like(sums)
        counts[...] = jnp.zeros_like(counts)

    ssum = p_ref[0] + p_ref[1] + hs2_ref[...]
    h2 = jnp.maximum(ssum * dinv_ref[...] + b2_ref[...], 0.0)
    b = batch_ref[0]
    oh = (lax.broadcasted_iota(jnp.int32, (_N_GRAPHS, blk), 0) == b
          ).astype(jnp.float32)
    sums[...] += jnp.dot(oh, h2, preferred_element_type=jnp.float32)
    counts[...] += jnp.sum(oh, axis=1, keepdims=True)

    @pl.when(i == n_blocks - 1)
    def _finish():
        pooled = sums[...] / jnp.maximum(counts[...], 1.0)
        g1 = jnp.maximum(
            jnp.dot(pooled, fc1w_ref[...],
                    preferred_element_type=jnp.float32) + fc1b_ref[...], 0.0)
        z = jnp.dot(g1, fc2w_ref[...],
                    preferred_element_type=jnp.float32) + fc2b_ref[...]
        out_ref[...] = jax.nn.sigmoid(z)


def kernel(x, edge_index, batch, W1, b1, W2, b2, fc1_W, fc1_b, fc2_W, fc2_b):
    n, d_in = x.shape
    n_edges = edge_index.shape[1]
    d1 = W1.shape[1]
    d2 = W2.shape[1]
    blk = 2000
    n_blocks = n // blk

    ntiles = _N_CORES * _N_SUB
    e_per_tile = n_edges // ntiles
    chunk1 = 400
    chunk2 = 1000
    src_flat = edge_index[0].astype(jnp.int32)
    dst_flat = edge_index[1].astype(jnp.int32)
    src1 = src_flat.reshape(ntiles, e_per_tile // chunk1, chunk1)
    dst1 = dst_flat.reshape(ntiles, e_per_tile // chunk1, chunk1)
    src2 = src_flat.reshape(ntiles, e_per_tile // chunk2, chunk2)
    dst2 = dst_flat.reshape(ntiles, e_per_tile // chunk2, chunk2)
    dstd = dst_flat.reshape(ntiles, e_per_tile // _DEG_CHUNK, _DEG_CHUNK)
    batch3d = batch.astype(jnp.int32).reshape(n // blk, 1, blk)

    n_pad = _N_SUB * ((n + 8 * _N_SUB - 1) // (8 * _N_SUB)) * 8
    ones16 = jnp.ones((_DEG_CHUNK, _DEG_W), jnp.float32)
    zeros16 = jnp.zeros((n_pad, _DEG_W), jnp.float32)
    zeros1 = jnp.zeros((n_pad, d1), jnp.float32)
    zeros2 = jnp.zeros((n_pad, d2), jnp.float32)

    degp = _make_sc_degree(n_pad, n_edges)(dstd, ones16, zeros16)

    dinv, hs1 = pl.pallas_call(
        _tc1_body,
        grid=(n_blocks,),
        in_specs=[
            pl.BlockSpec((_N_CORES, blk, _DEG_W), lambda i: (0, i, 0)),
            pl.BlockSpec((blk, d_in), lambda i: (i, 0)),
            pl.BlockSpec((d_in, d1), lambda i: (0, 0)),
        ],
        out_specs=[
            pl.BlockSpec((blk, 1), lambda i: (i, 0)),
            pl.BlockSpec((blk, d1), lambda i: (i, 0)),
        ],
        out_shape=[
            jax.ShapeDtypeStruct((n, 1), jnp.float32),
            jax.ShapeDtypeStruct((n, d1), jnp.float32),
        ],
    )(degp, x, W1)

    p1 = _make_sc_scatter(n_pad, n_edges, d1, chunk1)(src1, dst1, hs1, zeros1)

    hs2 = pl.pallas_call(
        _tc2_body,
        grid=(n_blocks,),
        in_specs=[
            pl.BlockSpec((_N_CORES, blk, d1), lambda i: (0, i, 0)),
            pl.BlockSpec((blk, d1), lambda i: (i, 0)),
            pl.BlockSpec((blk, 1), lambda i: (i, 0)),
            pl.BlockSpec((1, d1), lambda i: (0, 0)),
            pl.BlockSpec((d1, d2), lambda i: (0, 0)),
        ],
        out_specs=pl.BlockSpec((blk, d2), lambda i: (i, 0)),
        out_shape=jax.ShapeDtypeStruct((n, d2), jnp.float32),
    )(p1, hs1, dinv, b1.reshape(1, d1), W2)

    p2 = _make_sc_scatter(n_pad, n_edges, d2, chunk2)(src2, dst2, hs2, zeros2)

    d3 = fc1_W.shape[1]
    out = pl.pallas_call(
        functools.partial(_tc3_body, n_blocks=n_blocks, blk=blk),
        grid=(n_blocks,),
        in_specs=[
            pl.BlockSpec((_N_CORES, blk, d2), lambda i: (0, i, 0)),
            pl.BlockSpec((blk, d2), lambda i: (i, 0)),
            pl.BlockSpec((blk, 1), lambda i: (i, 0)),
            pl.BlockSpec((1, d2), lambda i: (0, 0)),
            pl.BlockSpec((1, 1, blk), lambda i: (i, 0, 0)),
            pl.BlockSpec((d2, d3), lambda i: (0, 0)),
            pl.BlockSpec((1, d3), lambda i: (0, 0)),
            pl.BlockSpec((d3, 1), lambda i: (0, 0)),
            pl.BlockSpec((1, 1), lambda i: (0, 0)),
        ],
        out_specs=pl.BlockSpec((_N_GRAPHS, 1), lambda i: (0, 0)),
        out_shape=jax.ShapeDtypeStruct((_N_GRAPHS, 1), jnp.float32),
        scratch_shapes=[
            pltpu.VMEM((_N_GRAPHS, d2), jnp.float32),
            pltpu.VMEM((_N_GRAPHS, 1), jnp.float32),
        ],
    )(p2, hs2, dinv, b2.reshape(1, d2), batch3d, fc1_W,
      fc1_b.reshape(1, d3), fc2_W, fc2_b.reshape(1, 1))

    return out


# speedup vs baseline: 1.0183x; 1.0183x over previous
import functools

import jax
import jax.numpy as jnp
from jax import lax
from jax.experimental import pallas as pl
from jax.experimental.pallas import tpu as pltpu
from jax.experimental.pallas import tpu_sc as plsc

_N_CORES = 2
_N_SUB = 16
_N_GRAPHS = 64
_DEG_CHUNK = 2000
_DEG_W = 16


def _make_sc_degree(n_pad, n_edges):
    ntiles = _N_CORES * _N_SUB
    e_per_tile = n_edges // ntiles
    n_chunks = e_per_tile // _DEG_CHUNK
    rps = n_pad // _N_SUB

    mesh = plsc.VectorSubcoreMesh(core_axis_name="c", subcore_axis_name="s")

    @functools.partial(
        pl.kernel,
        out_type=jax.ShapeDtypeStruct((_N_CORES, n_pad, _DEG_W), jnp.float32),
        mesh=mesh,
        compiler_params=pltpu.CompilerParams(use_tc_tiling_on_sc=False),
        scratch_types=[
            pltpu.VMEM((n_chunks, _DEG_CHUNK), jnp.int32),
            pltpu.VMEM((_DEG_CHUNK, _DEG_W), jnp.float32),
            pltpu.SemaphoreType.DMA,
            pltpu.VMEM_SHARED((n_pad, _DEG_W), jnp.float32),
        ],
    )
    def deg_kernel(dst_hbm, ones_hbm, zero_hbm, out_hbm, didx, ones_v, psem,
                   acc):
        c = lax.axis_index("c")
        s = lax.axis_index("s")
        tid = c * _N_SUB + s
        base = tid * e_per_tile

        def pre(i, carry):
            pltpu.async_copy(dst_hbm.at[pl.ds(base + i * _DEG_CHUNK,
                                              _DEG_CHUNK)], didx.at[i], psem)
            return carry

        lax.fori_loop(0, n_chunks, pre, 0)
        pltpu.sync_copy(ones_hbm, ones_v)

        def pre_drain(i, carry):
            pltpu.make_async_copy(dst_hbm.at[pl.ds(base, _DEG_CHUNK)],
                                  didx.at[i], psem).wait()
            return carry

        lax.fori_loop(0, n_chunks, pre_drain, 0)
        pltpu.sync_copy(zero_hbm.at[pl.ds(s * rps, rps)],
                        acc.at[pl.ds(s * rps, rps)])
        plsc.subcore_barrier()

        def body(i, carry):
            pltpu.sync_copy(ones_v, acc.at[didx.at[i]], add=True)
            return carry

        lax.fori_loop(0, n_chunks, body, 0)
        plsc.subcore_barrier()
        pltpu.sync_copy(acc.at[pl.ds(s * rps, rps)],
                        out_hbm.at[c, pl.ds(s * rps, rps)])

    return deg_kernel


def _make_sc_scatter(n_pad, n_edges, d, chunk):
    ntiles = _N_CORES * _N_SUB
    e_per_tile = n_edges // ntiles
    n_chunks = e_per_tile // chunk
    rps = n_pad // _N_SUB

    mesh = plsc.VectorSubcoreMesh(core_axis_name="c", subcore_axis_name="s")

    @functools.partial(
        pl.kernel,
        out_type=jax.ShapeDtypeStruct((_N_CORES, n_pad, d), jnp.float32),
        mesh=mesh,
        compiler_params=pltpu.CompilerParams(use_tc_tiling_on_sc=False),
        scratch_types=[
            pltpu.VMEM((n_chunks, chunk), jnp.int32),
            pltpu.VMEM((n_chunks, chunk), jnp.int32),
            pltpu.VMEM((2, chunk, d), jnp.float32),
            pltpu.SemaphoreType.DMA,
            pltpu.VMEM_SHARED((n_pad, d), jnp.float32),
            pltpu.SemaphoreType.DMA((2,)),
            pltpu.SemaphoreType.DMA((2,)),
        ],
    )
    def scatter_kernel(src_hbm, dst_hbm, feat_hbm, zero_hbm, out_hbm,
                       sidx, didx, rows, psem, acc, gsem, ssem):
        c = lax.axis_index("c")
        s = lax.axis_index("s")
        tid = c * _N_SUB + s
        base = tid * e_per_tile

        def pre(i, carry):
            off = base + i * chunk
            pltpu.async_copy(src_hbm.at[pl.ds(off, chunk)], sidx.at[i], psem)
            pltpu.async_copy(dst_hbm.at[pl.ds(off, chunk)], didx.at[i], psem)
            return carry

        lax.fori_loop(0, n_chunks, pre, 0)
        pltpu.sync_copy(zero_hbm.at[pl.ds(s * rps, rps)],
                        acc.at[pl.ds(s * rps, rps)])

        def pre_drain(i, carry):
            pltpu.make_async_copy(src_hbm.at[pl.ds(base, chunk)], sidx.at[i],
                                  psem).wait()
            pltpu.make_async_copy(dst_hbm.at[pl.ds(base, chunk)], didx.at[i],
                                  psem).wait()
            return carry

        lax.fori_loop(0, n_chunks, pre_drain, 0)
        plsc.subcore_barrier()
        pltpu.async_copy(feat_hbm.at[sidx.at[0]], rows.at[0], gsem.at[0])

        def body(i, carry):
            b = lax.rem(i, 2)
            nb = lax.rem(i + 1, 2)

            @pl.when(i >= 1)
            def _drain_prev_scatter():
                pltpu.make_async_copy(rows.at[nb], acc.at[didx.at[i - 1]],
                                      ssem.at[nb]).wait()

            @pl.when(i + 1 < n_chunks)
            def _fire_next_gather():
                pltpu.async_copy(feat_hbm.at[sidx.at[i + 1]], rows.at[nb],
                                 gsem.at[nb])

            pltpu.make_async_copy(feat_hbm.at[sidx.at[i]], rows.at[b],
                                  gsem.at[b]).wait()
            pltpu.async_copy(rows.at[b], acc.at[didx.at[i]], ssem.at[b],
                             add=True)
            return carry

        lax.fori_loop(0, n_chunks, body, 0)
        last = (n_chunks - 1) % 2
        pltpu.make_async_copy(rows.at[last], acc.at[didx.at[n_chunks - 1]],
                              ssem.at[last]).wait()
        plsc.subcore_barrier()
        pltpu.sync_copy(acc.at[pl.ds(s * rps, rps)],
                        out_hbm.at[c, pl.ds(s * rps, rps)])

    return scatter_kernel


def _tc1_body(degp_ref, x_ref, w1_ref, dinv_ref, hs_ref):
    deg = degp_ref[0][:, 0:1] + degp_ref[1][:, 0:1] + 1.0
    dinv = lax.rsqrt(deg)
    h = jnp.dot(x_ref[...], w1_ref[...], preferred_element_type=jnp.float32)
    dinv_ref[...] = dinv
    hs_ref[...] = h * dinv


def _tc2_body(p_ref, hs1_ref, dinv_ref, b1_ref, w2_ref, hs2_ref):
    ssum = p_ref[0] + p_ref[1] + hs1_ref[...]
    dinv = dinv_ref[...]
    h1 = jnp.maximum(ssum * dinv + b1_ref[...], 0.0)
    hs2_ref[...] = jnp.dot(h1, w2_ref[...],
                           preferred_element_type=jnp.float32) * dinv


def _tc3_body(p_ref, hs2_ref, dinv_ref, b2_ref, batch_ref, fc1w_ref,
              fc1b_ref, fc2w_ref, fc2b_ref, out_ref, sums, counts, *,
              n_blocks, blk):
    i = pl.program_id(0)

    @pl.when(i == 0)
    def _init():
        sums[...] = jnp.zeros_like(sums)
        counts[...] = jnp.zeros_like(counts)

    ssum = p_ref[0] + p_ref[1] + hs2_ref[...]
    h2 = jnp.maximum(ssum * dinv_ref[...] + b2_ref[...], 0.0)
    b = batch_ref[0]
    oh = (lax.broadcasted_iota(jnp.int32, (_N_GRAPHS, blk), 0) == b
          ).astype(jnp.float32)
    sums[...] += jnp.dot(oh, h2, preferred_element_type=jnp.float32)
    counts[...] += jnp.sum(oh, axis=1, keepdims=True)

    @pl.when(i == n_blocks - 1)
    def _finish():
        pooled = sums[...] / jnp.maximum(counts[...], 1.0)
        g1 = jnp.maximum(
            jnp.dot(pooled, fc1w_ref[...],
                    preferred_element_type=jnp.float32) + fc1b_ref[...], 0.0)
        z = jnp.dot(g1, fc2w_ref[...],
                    preferred_element_type=jnp.float32) + fc2b_ref[...]
        out_ref[...] = jax.nn.sigmoid(z)


def kernel(x, edge_index, batch, W1, b1, W2, b2, fc1_W, fc1_b, fc2_W, fc2_b):
    n, d_in = x.shape
    n_edges = edge_index.shape[1]
    d1 = W1.shape[1]
    d2 = W2.shape[1]
    blk = 2000
    n_blocks = n // blk

    chunk1 = 400
    chunk2 = 1000
    src = edge_index[0].astype(jnp.int32)
    dst = edge_index[1].astype(jnp.int32)
    batch3d = batch.astype(jnp.int32).reshape(n // blk, 1, blk)

    n_pad = _N_SUB * ((n + 8 * _N_SUB - 1) // (8 * _N_SUB)) * 8
    ones16 = jnp.ones((_DEG_CHUNK, _DEG_W), jnp.float32)
    zeros16 = jnp.zeros((n_pad, _DEG_W), jnp.float32)
    zeros1 = jnp.zeros((n_pad, d1), jnp.float32)
    zeros2 = jnp.zeros((n_pad, d2), jnp.float32)

    degp = _make_sc_degree(n_pad, n_edges)(dst, ones16, zeros16)

    dinv, hs1 = pl.pallas_call(
        _tc1_body,
        grid=(n_blocks,),
        in_specs=[
            pl.BlockSpec((_N_CORES, blk, _DEG_W), lambda i: (0, i, 0)),
            pl.BlockSpec((blk, d_in), lambda i: (i, 0)),
            pl.BlockSpec((d_in, d1), lambda i: (0, 0)),
        ],
        out_specs=[
            pl.BlockSpec((blk, 1), lambda i: (i, 0)),
            pl.BlockSpec((blk, d1), lambda i: (i, 0)),
        ],
        out_shape=[
            jax.ShapeDtypeStruct((n, 1), jnp.float32),
            jax.ShapeDtypeStruct((n, d1), jnp.float32),
        ],
    )(degp, x, W1)

    p1 = _make_sc_scatter(n_pad, n_edges, d1, chunk1)(src, dst, hs1, zeros1)

    hs2 = pl.pallas_call(
        _tc2_body,
        grid=(n_blocks,),
        in_specs=[
            pl.BlockSpec((_N_CORES, blk, d1), lambda i: (0, i, 0)),
            pl.BlockSpec((blk, d1), lambda i: (i, 0)),
            pl.BlockSpec((blk, 1), lambda i: (i, 0)),
            pl.BlockSpec((1, d1), lambda i: (0, 0)),
            pl.BlockSpec((d1, d2), lambda i: (0, 0)),
        ],
        out_specs=pl.BlockSpec((blk, d2), lambda i: (i, 0)),
        out_shape=jax.ShapeDtypeStruct((n, d2), jnp.float32),
    )(p1, hs1, dinv, b1.reshape(1, d1), W2)

    p2 = _make_sc_scatter(n_pad, n_edges, d2, chunk2)(src, dst, hs2, zeros2)

    d3 = fc1_W.shape[1]
    out = pl.pallas_call(
        functools.partial(_tc3_body, n_blocks=n_blocks, blk=blk),
        grid=(n_blocks,),
        in_specs=[
            pl.BlockSpec((_N_CORES, blk, d2), lambda i: (0, i, 0)),
            pl.BlockSpec((blk, d2), lambda i: (i, 0)),
            pl.BlockSpec((blk, 1), lambda i: (i, 0)),
            pl.BlockSpec((1, d2), lambda i: (0, 0)),
            pl.BlockSpec((1, 1, blk), lambda i: (i, 0, 0)),
            pl.BlockSpec((d2, d3), lambda i: (0, 0)),
            pl.BlockSpec((1, d3), lambda i: (0, 0)),
            pl.BlockSpec((d3, 1), lambda i: (0, 0)),
            pl.BlockSpec((1, 1), lambda i: (0, 0)),
        ],
        out_specs=pl.BlockSpec((_N_GRAPHS, 1), lambda i: (0, 0)),
        out_shape=jax.ShapeDtypeStruct((_N_GRAPHS, 1), jnp.float32),
        scratch_shapes=[
            pltpu.VMEM((_N_GRAPHS, d2), jnp.float32),
            pltpu.VMEM((_N_GRAPHS, 1), jnp.float32),
        ],
    )(p2, hs2, dinv, b2.reshape(1, d2), batch3d, fc1_W,
      fc1_b.reshape(1, d3), fc2_W, fc2_b.reshape(1, 1))

    return out


# speedup vs baseline: 1.0810x; 1.0616x over previous
import functools

import jax
import jax.numpy as jnp
from jax import lax
from jax.experimental import pallas as pl
from jax.experimental.pallas import tpu as pltpu
from jax.experimental.pallas import tpu_sc as plsc

_N_CORES = 2
_N_SUB = 16
_N_GRAPHS = 64
_DEG_CHUNK = 2000
_DEG_W = 16


def _make_sc_degree(n_pad, n_edges):
    ntiles = _N_CORES * _N_SUB
    e_per_tile = n_edges // ntiles
    n_chunks = e_per_tile // _DEG_CHUNK
    rps = n_pad // _N_SUB

    mesh = plsc.VectorSubcoreMesh(core_axis_name="c", subcore_axis_name="s")

    @functools.partial(
        pl.kernel,
        out_type=jax.ShapeDtypeStruct((_N_CORES, n_pad, _DEG_W), jnp.float32),
        mesh=mesh,
        compiler_params=pltpu.CompilerParams(use_tc_tiling_on_sc=False),
        scratch_types=[
            pltpu.VMEM((n_chunks, _DEG_CHUNK), jnp.int32),
            pltpu.VMEM((_DEG_CHUNK, _DEG_W), jnp.float32),
            pltpu.SemaphoreType.DMA,
            pltpu.VMEM_SHARED((n_pad, _DEG_W), jnp.float32),
        ],
    )
    def deg_kernel(ei_hbm, ones_hbm, zero_hbm, out_hbm, didx, ones_v, psem,
                   acc):
        c = lax.axis_index("c")
        s = lax.axis_index("s")
        tid = c * _N_SUB + s
        base = tid * e_per_tile

        def pre(i, carry):
            pltpu.async_copy(ei_hbm.at[1, pl.ds(base + i * _DEG_CHUNK,
                                                _DEG_CHUNK)], didx.at[i], psem)
            return carry

        lax.fori_loop(0, n_chunks, pre, 0)
        pltpu.sync_copy(ones_hbm, ones_v)

        def pre_drain(i, carry):
            pltpu.make_async_copy(ei_hbm.at[1, pl.ds(base, _DEG_CHUNK)],
                                  didx.at[i], psem).wait()
            return carry

        lax.fori_loop(0, n_chunks, pre_drain, 0)
        pltpu.sync_copy(zero_hbm.at[pl.ds(s * rps, rps)],
                        acc.at[pl.ds(s * rps, rps)])
        plsc.subcore_barrier()

        def body(i, carry):
            pltpu.sync_copy(ones_v, acc.at[didx.at[i]], add=True)
            return carry

        lax.fori_loop(0, n_chunks, body, 0)
        plsc.subcore_barrier()
        pltpu.sync_copy(acc.at[pl.ds(s * rps, rps)],
                        out_hbm.at[c, pl.ds(s * rps, rps)])

    return deg_kernel


def _make_sc_scatter(n_pad, n_edges, d, chunk, nbuf):
    ntiles = _N_CORES * _N_SUB
    e_per_tile = n_edges // ntiles
    n_chunks = e_per_tile // chunk
    rps = n_pad // _N_SUB

    mesh = plsc.VectorSubcoreMesh(core_axis_name="c", subcore_axis_name="s")

    @functools.partial(
        pl.kernel,
        out_type=jax.ShapeDtypeStruct((_N_CORES, n_pad, d), jnp.float32),
        mesh=mesh,
        compiler_params=pltpu.CompilerParams(use_tc_tiling_on_sc=False),
        scratch_types=[
            pltpu.VMEM((n_chunks, chunk), jnp.int32),
            pltpu.VMEM((n_chunks, chunk), jnp.int32),
            pltpu.VMEM((nbuf, chunk, d), jnp.float32),
            pltpu.SemaphoreType.DMA,
            pltpu.VMEM_SHARED((n_pad, d), jnp.float32),
            pltpu.SemaphoreType.DMA((nbuf,)),
            pltpu.SemaphoreType.DMA((nbuf,)),
        ],
    )
    def scatter_kernel(ei_hbm, feat_hbm, zero_hbm, out_hbm,
                       sidx, didx, rows, psem, acc, gsem, ssem):
        c = lax.axis_index("c")
        s = lax.axis_index("s")
        tid = c * _N_SUB + s
        base = tid * e_per_tile

        def pre(i, carry):
            off = base + i * chunk
            pltpu.async_copy(ei_hbm.at[0, pl.ds(off, chunk)], sidx.at[i], psem)
            pltpu.async_copy(ei_hbm.at[1, pl.ds(off, chunk)], didx.at[i], psem)
            return carry

        lax.fori_loop(0, n_chunks, pre, 0)
        pltpu.sync_copy(zero_hbm.at[pl.ds(s * rps, rps)],
                        acc.at[pl.ds(s * rps, rps)])

        def pre_drain(i, carry):
            pltpu.make_async_copy(ei_hbm.at[0, pl.ds(base, chunk)], sidx.at[i],
                                  psem).wait()
            pltpu.make_async_copy(ei_hbm.at[1, pl.ds(base, chunk)], didx.at[i],
                                  psem).wait()
            return carry

        lax.fori_loop(0, n_chunks, pre_drain, 0)
        plsc.subcore_barrier()
        for k in range(min(nbuf - 1, n_chunks)):
            pltpu.async_copy(feat_hbm.at[sidx.at[k]], rows.at[k], gsem.at[k])

        def body(i, carry):
            b = lax.rem(i, nbuf)
            pltpu.make_async_copy(feat_hbm.at[sidx.at[i]], rows.at[b],
                                  gsem.at[b]).wait()
            pltpu.async_copy(rows.at[b], acc.at[didx.at[i]], ssem.at[b],
                             add=True)
            j = i + nbuf - 1
            jb = lax.rem(j, nbuf)

            @pl.when(j < n_chunks)
            def _fire_ahead():
                @pl.when(i >= 1)
                def _drain():
                    pltpu.make_async_copy(rows.at[jb], acc.at[didx.at[i - 1]],
                                          ssem.at[jb]).wait()

                pltpu.async_copy(feat_hbm.at[sidx.at[j]], rows.at[jb],
                                 gsem.at[jb])

            return carry

        lax.fori_loop(0, n_chunks, body, 0)

        def tail(i, carry):
            t = n_chunks - nbuf + i
            tb = lax.rem(t, nbuf)

            @pl.when(t >= 0)
            def _():
                pltpu.make_async_copy(rows.at[tb], acc.at[didx.at[t]],
                                      ssem.at[tb]).wait()

            return carry

        lax.fori_loop(0, nbuf, tail, 0)
        plsc.subcore_barrier()
        pltpu.sync_copy(acc.at[pl.ds(s * rps, rps)],
                        out_hbm.at[c, pl.ds(s * rps, rps)])

    return scatter_kernel


def _tc1_body(degp_ref, x_ref, w1_ref, dinv_ref, hs_ref):
    deg = degp_ref[0][:, 0:1] + degp_ref[1][:, 0:1] + 1.0
    dinv = lax.rsqrt(deg)
    h = jnp.dot(x_ref[...], w1_ref[...], preferred_element_type=jnp.float32)
    dinv_ref[...] = dinv
    hs_ref[...] = h * dinv


def _tc2_body(p_ref, hs1_ref, dinv_ref, b1_ref, w2_ref, hs2_ref):
    ssum = p_ref[0] + p_ref[1] + hs1_ref[...]
    dinv = dinv_ref[...]
    h1 = jnp.maximum(ssum * dinv + b1_ref[...], 0.0)
    hs2_ref[...] = jnp.dot(h1, w2_ref[...],
                           preferred_element_type=jnp.float32) * dinv


def _tc3_body(p_ref, hs2_ref, dinv_ref, b2_ref, batch_ref, fc1w_ref,
              fc1b_ref, fc2w_ref, fc2b_ref, out_ref, sums, counts, *,
              n_blocks, blk):
    i = pl.program_id(0)

    @pl.when(i == 0)
    def _init():
        sums[...] = jnp.zeros_like(sums)
        counts[...] = jnp.zeros_like(counts)

    ssum = p_ref[0] + p_ref[1] + hs2_ref[...]
    h2 = jnp.maximum(ssum * dinv_ref[...] + b2_ref[...], 0.0)
    b = batch_ref[0]
    oh = (lax.broadcasted_iota(jnp.int32, (_N_GRAPHS, blk), 0) == b
          ).astype(jnp.float32)
    sums[...] += jnp.dot(oh, h2, preferred_element_type=jnp.float32)
    counts[...] += jnp.sum(oh, axis=1, keepdims=True)

    @pl.when(i == n_blocks - 1)
    def _finish():
        pooled = sums[...] / jnp.maximum(counts[...], 1.0)
        g1 = jnp.maximum(
            jnp.dot(pooled, fc1w_ref[...],
                    preferred_element_type=jnp.float32) + fc1b_ref[...], 0.0)
        z = jnp.dot(g1, fc2w_ref[...],
                    preferred_element_type=jnp.float32) + fc2b_ref[...]
        out_ref[...] = jax.nn.sigmoid(z)


def kernel(x, edge_index, batch, W1, b1, W2, b2, fc1_W, fc1_b, fc2_W, fc2_b):
    n, d_in = x.shape
    n_edges = edge_index.shape[1]
    d1 = W1.shape[1]
    d2 = W2.shape[1]
    blk = 2000
    n_blocks = n // blk

    chunk1 = 200
    chunk2 = 400
    ei = edge_index.astype(jnp.int32)
    batch3d = batch.astype(jnp.int32).reshape(n // blk, 1, blk)

    n_pad = _N_SUB * ((n + 8 * _N_SUB - 1) // (8 * _N_SUB)) * 8
    ones16 = jnp.ones((_DEG_CHUNK, _DEG_W), jnp.float32)
    zeros16 = jnp.zeros((n_pad, _DEG_W), jnp.float32)
    zeros1 = jnp.zeros((n_pad, d1), jnp.float32)
    zeros2 = jnp.zeros((n_pad, d2), jnp.float32)

    degp = _make_sc_degree(n_pad, n_edges)(ei, ones16, zeros16)

    dinv, hs1 = pl.pallas_call(
        _tc1_body,
        grid=(n_blocks,),
        in_specs=[
            pl.BlockSpec((_N_CORES, blk, _DEG_W), lambda i: (0, i, 0)),
            pl.BlockSpec((blk, d_in), lambda i: (i, 0)),
            pl.BlockSpec((d_in, d1), lambda i: (0, 0)),
        ],
        out_specs=[
            pl.BlockSpec((blk, 1), lambda i: (i, 0)),
            pl.BlockSpec((blk, d1), lambda i: (i, 0)),
        ],
        out_shape=[
            jax.ShapeDtypeStruct((n, 1), jnp.float32),
            jax.ShapeDtypeStruct((n, d1), jnp.float32),
        ],
    )(degp, x, W1)

    p1 = _make_sc_scatter(n_pad, n_edges, d1, chunk1, 4)(ei, hs1, zeros1)

    hs2 = pl.pallas_call(
        _tc2_body,
        grid=(n_blocks,),
        in_specs=[
            pl.BlockSpec((_N_CORES, blk, d1), lambda i: (0, i, 0)),
            pl.BlockSpec((blk, d1), lambda i: (i, 0)),
            pl.BlockSpec((blk, 1), lambda i: (i, 0)),
            pl.BlockSpec((1, d1), lambda i: (0, 0)),
            pl.BlockSpec((d1, d2), lambda i: (0, 0)),
        ],
        out_specs=pl.BlockSpec((blk, d2), lambda i: (i, 0)),
        out_shape=jax.ShapeDtypeStruct((n, d2), jnp.float32),
    )(p1, hs1, dinv, b1.reshape(1, d1), W2)

    p2 = _make_sc_scatter(n_pad, n_edges, d2, chunk2, 4)(ei, hs2, zeros2)

    d3 = fc1_W.shape[1]
    out = pl.pallas_call(
        functools.partial(_tc3_body, n_blocks=n_blocks, blk=blk),
        grid=(n_blocks,),
        in_specs=[
            pl.BlockSpec((_N_CORES, blk, d2), lambda i: (0, i, 0)),
            pl.BlockSpec((blk, d2), lambda i: (i, 0)),
            pl.BlockSpec((blk, 1), lambda i: (i, 0)),
            pl.BlockSpec((1, d2), lambda i: (0, 0)),
            pl.BlockSpec((1, 1, blk), lambda i: (i, 0, 0)),
            pl.BlockSpec((d2, d3), lambda i: (0, 0)),
            pl.BlockSpec((1, d3), lambda i: (0, 0)),
            pl.BlockSpec((d3, 1), lambda i: (0, 0)),
            pl.BlockSpec((1, 1), lambda i: (0, 0)),
        ],
        out_specs=pl.BlockSpec((_N_GRAPHS, 1), lambda i: (0, 0)),
        out_shape=jax.ShapeDtypeStruct((_N_GRAPHS, 1), jnp.float32),
        scratch_shapes=[
            pltpu.VMEM((_N_GRAPHS, d2), jnp.float32),
            pltpu.VMEM((_N_GRAPHS, 1), jnp.float32),
        ],
    )(p2, hs2, dinv, b2.reshape(1, d2), batch3d, fc1_W,
      fc1_b.reshape(1, d3), fc2_W, fc2_b.reshape(1, 1))

    return out


# speedup vs baseline: 1.2015x; 1.1115x over previous
import functools

import jax
import jax.numpy as jnp
from jax import lax
from jax.experimental import pallas as pl
from jax.experimental.pallas import tpu as pltpu
from jax.experimental.pallas import tpu_sc as plsc

_N_CORES = 2
_N_SUB = 16
_N_GRAPHS = 64
_DEG_CHUNK = 2000
_DEG_W = 16


def _make_sc_degree(n_pad, n_edges):
    ntiles = _N_CORES * _N_SUB
    e_per_tile = n_edges // ntiles
    n_chunks = e_per_tile // _DEG_CHUNK
    rps = n_pad // _N_SUB

    mesh = plsc.VectorSubcoreMesh(core_axis_name="c", subcore_axis_name="s")

    @functools.partial(
        pl.kernel,
        out_type=jax.ShapeDtypeStruct((_N_CORES, n_pad, 128), jnp.float32),
        mesh=mesh,
        compiler_params=pltpu.CompilerParams(use_tc_tiling_on_sc=False),
        scratch_types=[
            pltpu.VMEM((n_chunks, _DEG_CHUNK), jnp.int32),
            pltpu.VMEM((_DEG_CHUNK, _DEG_W), jnp.float32),
            pltpu.SemaphoreType.DMA,
            pltpu.VMEM_SHARED((n_pad, _DEG_W), jnp.float32),
        ],
    )
    def deg_kernel(ei_hbm, ones_hbm, zero_hbm, out_hbm, didx, ones_v, psem,
                   acc):
        c = lax.axis_index("c")
        s = lax.axis_index("s")
        tid = c * _N_SUB + s
        base = tid * e_per_tile

        def pre(i, carry):
            pltpu.async_copy(ei_hbm.at[1, pl.ds(base + i * _DEG_CHUNK,
                                                _DEG_CHUNK)], didx.at[i], psem)
            return carry

        lax.fori_loop(0, n_chunks, pre, 0)
        pltpu.sync_copy(ones_hbm, ones_v)

        def pre_drain(i, carry):
            pltpu.make_async_copy(ei_hbm.at[1, pl.ds(base, _DEG_CHUNK)],
                                  didx.at[i], psem).wait()
            return carry

        lax.fori_loop(0, n_chunks, pre_drain, 0)
        pltpu.sync_copy(zero_hbm.at[pl.ds(s * rps, rps)],
                        acc.at[pl.ds(s * rps, rps)])
        plsc.subcore_barrier()

        def body(i, carry):
            pltpu.sync_copy(ones_v, acc.at[didx.at[i]], add=True)
            return carry

        lax.fori_loop(0, n_chunks, body, 0)
        plsc.subcore_barrier()
        pltpu.sync_copy(acc.at[pl.ds(s * rps, rps)],
                        out_hbm.at[c, pl.ds(s * rps, rps), pl.ds(0, _DEG_W)])

    return deg_kernel


def _make_sc_scatter(n_pad, n_edges, d, chunk, nbuf):
    ntiles = _N_CORES * _N_SUB
    e_per_tile = n_edges // ntiles
    n_chunks = e_per_tile // chunk
    rps = n_pad // _N_SUB

    mesh = plsc.VectorSubcoreMesh(core_axis_name="c", subcore_axis_name="s")

    @functools.partial(
        pl.kernel,
        out_type=jax.ShapeDtypeStruct((_N_CORES, n_pad, 128), jnp.float32),
        mesh=mesh,
        compiler_params=pltpu.CompilerParams(use_tc_tiling_on_sc=False),
        scratch_types=[
            pltpu.VMEM((n_chunks, chunk), jnp.int32),
            pltpu.VMEM((n_chunks, chunk), jnp.int32),
            pltpu.VMEM((nbuf, chunk, d), jnp.float32),
            pltpu.SemaphoreType.DMA,
            pltpu.VMEM_SHARED((n_pad, d), jnp.float32),
            pltpu.SemaphoreType.DMA((nbuf,)),
            pltpu.SemaphoreType.DMA((nbuf,)),
        ],
    )
    def scatter_kernel(ei_hbm, feat_hbm, zero_hbm, out_hbm,
                       sidx, didx, rows, psem, acc, gsem, ssem):
        c = lax.axis_index("c")
        s = lax.axis_index("s")
        tid = c * _N_SUB + s
        base = tid * e_per_tile

        def pre(i, carry):
            off = base + i * chunk
            pltpu.async_copy(ei_hbm.at[0, pl.ds(off, chunk)], sidx.at[i], psem)
            pltpu.async_copy(ei_hbm.at[1, pl.ds(off, chunk)], didx.at[i], psem)
            return carry

        lax.fori_loop(0, n_chunks, pre, 0)
        pltpu.sync_copy(zero_hbm.at[pl.ds(s * rps, rps)],
                        acc.at[pl.ds(s * rps, rps)])

        def pre_drain(i, carry):
            pltpu.make_async_copy(ei_hbm.at[0, pl.ds(base, chunk)], sidx.at[i],
                                  psem).wait()
            pltpu.make_async_copy(ei_hbm.at[1, pl.ds(base, chunk)], didx.at[i],
                                  psem).wait()
            return carry

        lax.fori_loop(0, n_chunks, pre_drain, 0)
        plsc.subcore_barrier()
        for k in range(min(nbuf - 1, n_chunks)):
            pltpu.async_copy(feat_hbm.at[sidx.at[k]], rows.at[k], gsem.at[k])

        def body(i, carry):
            b = lax.rem(i, nbuf)
            pltpu.make_async_copy(feat_hbm.at[sidx.at[i]], rows.at[b],
                                  gsem.at[b]).wait()
            pltpu.async_copy(rows.at[b], acc.at[didx.at[i]], ssem.at[b],
                             add=True)
            j = i + nbuf - 1
            jb = lax.rem(j, nbuf)

            @pl.when(j < n_chunks)
            def _fire_ahead():
                @pl.when(i >= 1)
                def _drain():
                    pltpu.make_async_copy(rows.at[jb], acc.at[didx.at[i - 1]],
                                          ssem.at[jb]).wait()

                pltpu.async_copy(feat_hbm.at[sidx.at[j]], rows.at[jb],
                                 gsem.at[jb])

            return carry

        lax.fori_loop(0, n_chunks, body, 0)

        def tail(i, carry):
            t = n_chunks - nbuf + i
            tb = lax.rem(t, nbuf)

            @pl.when(t >= 0)
            def _():
                pltpu.make_async_copy(rows.at[tb], acc.at[didx.at[t]],
                                      ssem.at[tb]).wait()

            return carry

        lax.fori_loop(0, nbuf, tail, 0)
        plsc.subcore_barrier()
        pltpu.sync_copy(acc.at[pl.ds(s * rps, rps)],
                        out_hbm.at[c, pl.ds(s * rps, rps), pl.ds(0, d)])

    return scatter_kernel


def _tc1_body(degp_ref, x_ref, w1_ref, dinv_ref, hs_ref):
    deg = degp_ref[0][:, 0:1] + degp_ref[1][:, 0:1] + 1.0
    dinv = lax.rsqrt(deg)
    h = jnp.dot(x_ref[...], w1_ref[...], preferred_element_type=jnp.float32)
    dinv_ref[...] = dinv
    hs_ref[...] = h * dinv


def _tc2_body(p_ref, hs1_ref, dinv_ref, b1_ref, w2_ref, hs2_ref, *, d):
    ssum = p_ref[0][:, 0:d] + p_ref[1][:, 0:d] + hs1_ref[...]
    dinv = dinv_ref[...]
    h1 = jnp.maximum(ssum * dinv + b1_ref[...], 0.0)
    hs2_ref[...] = jnp.dot(h1, w2_ref[...],
                           preferred_element_type=jnp.float32) * dinv


def _tc3_body(p_ref, hs2_ref, dinv_ref, b2_ref, batch_ref, fc1w_ref,
              fc1b_ref, fc2w_ref, fc2b_ref, out_ref, sums, counts, *,
              n_blocks, blk, d2):
    i = pl.program_id(0)

    @pl.when(i == 0)
    def _init():
        sums[...] = jnp.zeros_like(sums)
        counts[...] = jnp.zeros_like(counts)

    ssum = p_ref[0][:, 0:d2] + p_ref[1][:, 0:d2] + hs2_ref[...]
    h2 = jnp.maximum(ssum * dinv_ref[...] + b2_ref[...], 0.0)
    b = batch_ref[0]
    oh = (lax.broadcasted_iota(jnp.int32, (_N_GRAPHS, blk), 0) == b
          ).astype(jnp.float32)
    sums[...] += jnp.dot(oh, h2, preferred_element_type=jnp.float32)
    counts[...] += jnp.sum(oh, axis=1, keepdims=True)

    @pl.when(i == n_blocks - 1)
    def _finish():
        pooled = sums[...] / jnp.maximum(counts[...], 1.0)
        g1 = jnp.maximum(
            jnp.dot(pooled, fc1w_ref[...],
                    preferred_element_type=jnp.float32) + fc1b_ref[...], 0.0)
        z = jnp.dot(g1, fc2w_ref[...],
                    preferred_element_type=jnp.float32) + fc2b_ref[...]
        out_ref[...] = jax.nn.sigmoid(z)


def kernel(x, edge_index, batch, W1, b1, W2, b2, fc1_W, fc1_b, fc2_W, fc2_b):
    n, d_in = x.shape
    n_edges = edge_index.shape[1]
    d1 = W1.shape[1]
    d2 = W2.shape[1]
    blk = 2000
    n_blocks = n // blk

    chunk1 = 200
    chunk2 = 400
    ei = edge_index.astype(jnp.int32)
    batch3d = batch.astype(jnp.int32).reshape(n // blk, 1, blk)

    n_pad = _N_SUB * ((n + 8 * _N_SUB - 1) // (8 * _N_SUB)) * 8
    ones16 = jnp.ones((_DEG_CHUNK, _DEG_W), jnp.float32)
    zeros16 = jnp.zeros((n_pad, _DEG_W), jnp.float32)
    zeros1 = jnp.zeros((n_pad, d1), jnp.float32)
    zeros2 = jnp.zeros((n_pad, d2), jnp.float32)

    degp = _make_sc_degree(n_pad, n_edges)(ei, ones16, zeros16)

    dinv, hs1 = pl.pallas_call(
        _tc1_body,
        grid=(n_blocks,),
        in_specs=[
            pl.BlockSpec((_N_CORES, blk, 128), lambda i: (0, i, 0)),
            pl.BlockSpec((blk, d_in), lambda i: (i, 0)),
            pl.BlockSpec((d_in, d1), lambda i: (0, 0)),
        ],
        out_specs=[
            pl.BlockSpec((blk, 1), lambda i: (i, 0)),
            pl.BlockSpec((blk, d1), lambda i: (i, 0)),
        ],
        out_shape=[
            jax.ShapeDtypeStruct((n, 1), jnp.float32),
            jax.ShapeDtypeStruct((n, d1), jnp.float32),
        ],
    )(degp, x, W1)

    p1 = _make_sc_scatter(n_pad, n_edges, d1, chunk1, 4)(ei, hs1, zeros1)

    hs2 = pl.pallas_call(
        functools.partial(_tc2_body, d=d1),
        grid=(n_blocks,),
        in_specs=[
            pl.BlockSpec((_N_CORES, blk, 128), lambda i: (0, i, 0)),
            pl.BlockSpec((blk, d1), lambda i: (i, 0)),
            pl.BlockSpec((blk, 1), lambda i: (i, 0)),
            pl.BlockSpec((1, d1), lambda i: (0, 0)),
            pl.BlockSpec((d1, d2), lambda i: (0, 0)),
        ],
        out_specs=pl.BlockSpec((blk, d2), lambda i: (i, 0)),
        out_shape=jax.ShapeDtypeStruct((n, d2), jnp.float32),
    )(p1, hs1, dinv, b1.reshape(1, d1), W2)

    p2 = _make_sc_scatter(n_pad, n_edges, d2, chunk2, 4)(ei, hs2, zeros2)

    d3 = fc1_W.shape[1]
    out = pl.pallas_call(
        functools.partial(_tc3_body, n_blocks=n_blocks, blk=blk, d2=d2),
        grid=(n_blocks,),
        in_specs=[
            pl.BlockSpec((_N_CORES, blk, 128), lambda i: (0, i, 0)),
            pl.BlockSpec((blk, d2), lambda i: (i, 0)),
            pl.BlockSpec((blk, 1), lambda i: (i, 0)),
            pl.BlockSpec((1, d2), lambda i: (0, 0)),
            pl.BlockSpec((1, 1, blk), lambda i: (i, 0, 0)),
            pl.BlockSpec((d2, d3), lambda i: (0, 0)),
            pl.BlockSpec((1, d3), lambda i: (0, 0)),
            pl.BlockSpec((d3, 1), lambda i: (0, 0)),
            pl.BlockSpec((1, 1), lambda i: (0, 0)),
        ],
        out_specs=pl.BlockSpec((_N_GRAPHS, 1), lambda i: (0, 0)),
        out_shape=jax.ShapeDtypeStruct((_N_GRAPHS, 1), jnp.float32),
        scratch_shapes=[
            pltpu.VMEM((_N_GRAPHS, d2), jnp.float32),
            pltpu.VMEM((_N_GRAPHS, 1), jnp.float32),
        ],
    )(p2, hs2, dinv, b2.reshape(1, d2), batch3d, fc1_W,
      fc1_b.reshape(1, d3), fc2_W, fc2_b.reshape(1, 1))

    return out
